# Initial kernel scaffold; baseline (speedup 1.0000x reference)
import jax
import jax.numpy as jnp
from jax.experimental import pallas as pl


def kernel(input_ids, attention_mask, table):
    raise NotImplementedError("write your pallas kernel here")


# speedup vs baseline: 3.2931x; 3.2931x over previous
import functools

import jax
import jax.numpy as jnp
from jax import lax
from jax.experimental import pallas as pl
from jax.experimental.pallas import tpu as pltpu
from jax.experimental.pallas import tpu_sc as plsc

EMBED = 64


@functools.cache
def _make_gather(B: int):
    info = plsc.get_sparse_core_info()
    NC, NS = info.num_cores, info.num_subcores
    NW = NC * NS
    b_per_w = B // NW
    C = 128
    n_chunks = b_per_w // C
    mesh = plsc.VectorSubcoreMesh(core_axis_name="c", subcore_axis_name="s")

    @functools.partial(
        pl.kernel,
        mesh=mesh,
        out_type=jax.ShapeDtypeStruct((B, EMBED), jnp.float32),
        scratch_types=[
            pltpu.VMEM((C,), jnp.int32),
            pltpu.VMEM((C, EMBED), jnp.float32),
            pltpu.SemaphoreType.DMA,
        ],
        compiler_params=pltpu.CompilerParams(use_tc_tiling_on_sc=False),
    )
    def gather_kernel(idx_hbm, table_hbm, out_hbm, idx_v, rows_v, sem):
        wid = lax.axis_index("s") * NC + lax.axis_index("c")
        base = wid * b_per_w

        def body(i, carry):
            b0 = base + i * C
            pltpu.sync_copy(idx_hbm.at[pl.ds(b0, C)], idx_v)
            pltpu.async_copy(table_hbm.at[idx_v], rows_v, sem).wait()
            pltpu.sync_copy(rows_v, out_hbm.at[pl.ds(b0, C)])
            return carry

        lax.fori_loop(0, n_chunks, body, 0)

    return gather_kernel


def kernel(input_ids, attention_mask, table):
    B, S = input_ids.shape
    ids_flat = input_ids.reshape(B * S).astype(jnp.int32)
    rows = _make_gather(B * S)(ids_flat, table)
    return rows.reshape(B, S, EMBED), attention_mask


# speedup vs baseline: 4.4736x; 1.3585x over previous
import functools

import jax
import jax.numpy as jnp
from jax import lax
from jax.experimental import pallas as pl
from jax.experimental.pallas import tpu as pltpu
from jax.experimental.pallas import tpu_sc as plsc

EMBED = 64


@functools.cache
def _make_gather(B: int):
    info = plsc.get_sparse_core_info()
    NC, NS = info.num_cores, info.num_subcores
    NW = NC * NS
    b_per_w = B // NW
    C = 128
    NBUF = 4
    n_chunks = b_per_w // C
    n_groups = n_chunks // NBUF
    mesh = plsc.VectorSubcoreMesh(core_axis_name="c", subcore_axis_name="s")

    scratch = [
        pltpu.VMEM((NBUF, C), jnp.int32),
        pltpu.VMEM((NBUF, C, EMBED), jnp.float32),
    ] + [pltpu.SemaphoreType.DMA] * (3 * NBUF)

    @functools.partial(
        pl.kernel,
        mesh=mesh,
        out_type=jax.ShapeDtypeStruct((B, EMBED), jnp.float32),
        scratch_types=scratch,
        compiler_params=pltpu.CompilerParams(use_tc_tiling_on_sc=False),
    )
    def gather_kernel(idx_hbm, table_hbm, out_hbm, idx_v, rows_v, *sems):
        isem = sems[:NBUF]
        gsem = sems[NBUF:2 * NBUF]
        osem = sems[2 * NBUF:]
        wid = lax.axis_index("s") * NC + lax.axis_index("c")
        base = wid * b_per_w

        def start_idx(g, b):
            off = base + (g * NBUF + b) * C
            pltpu.async_copy(idx_hbm.at[pl.ds(off, C)], idx_v.at[b], isem[b])

        def wait_idx(b):
            pltpu.make_async_copy(
                idx_hbm.at[pl.ds(0, C)], idx_v.at[b], isem[b]).wait()

        def start_gather(b):
            pltpu.async_copy(table_hbm.at[idx_v.at[b]], rows_v.at[b], gsem[b])

        def wait_gather(b):
            pltpu.make_async_copy(
                table_hbm.at[idx_v.at[b]], rows_v.at[b], gsem[b]).wait()

        def start_out(g, b):
            off = base + (g * NBUF + b) * C
            pltpu.async_copy(rows_v.at[b], out_hbm.at[pl.ds(off, C)], osem[b])

        def wait_out(b):
            pltpu.make_async_copy(
                rows_v.at[b], out_hbm.at[pl.ds(0, C)], osem[b]).wait()

        for b in range(NBUF):
            start_idx(0, b)

        def group(g, carry):
            for b in range(NBUF):
                wait_idx(b)

                @pl.when(g > 0)
                def _():
                    wait_out(b)

                start_gather(b)
            for b in range(NBUF):
                wait_gather(b)
                start_out(g, b)

                @pl.when(g < n_groups - 1)
                def _():
                    start_idx(g + 1, b)

            return carry

        lax.fori_loop(0, n_groups, group, 0)
        for b in range(NBUF):
            wait_out(b)

    return gather_kernel


def kernel(input_ids, attention_mask, table):
    B, S = input_ids.shape
    ids_flat = input_ids.reshape(B * S).astype(jnp.int32)
    rows = _make_gather(B * S)(ids_flat, table)
    return rows.reshape(B, S, EMBED), attention_mask
